# baseline (speedup 1.0000x reference)
import functools

import jax
import jax.numpy as jnp
from jax import lax
from jax.experimental import pallas as pl
from jax.experimental.pallas import tpu as pltpu
from jax.experimental.pallas import tpu_sc as plsc

D_MODEL = 64
SCALE = 8.0
LANES = 16
CHUNK = 256
DCHUNK = 400


@jax.jit
def _emb_call(xf, lut):
    n = xf.shape[0]
    v = lut.shape[0]
    info = plsc.get_sparse_core_info()
    nw = info.num_cores * info.num_subcores
    n_per_w = n // nw
    n_chunks = n_per_w // CHUNK
    v_blocks = v // DCHUNK
    v_iters = -(-v_blocks // nw)

    mesh = plsc.VectorSubcoreMesh(core_axis_name="c", subcore_axis_name="s")

    @functools.partial(
        pl.kernel,
        mesh=mesh,
        out_type=jax.ShapeDtypeStruct((v, 128), jnp.float32),
        scratch_types=[
            pltpu.VMEM((DCHUNK, D_MODEL), jnp.float32),
            pltpu.VMEM((DCHUNK, 128), jnp.float32),
        ],
    )
    def detile(lut_hbm, wide_hbm, buf64_v, buf_v):
        wid = lax.axis_index("s") * info.num_cores + lax.axis_index("c")

        def chunk_body(i, carry):
            blk = wid + i * nw
            off = pl.multiple_of(blk * DCHUNK, 8)
            pltpu.sync_copy(lut_hbm.at[pl.ds(off, DCHUNK)], buf64_v)

            def widen_row(r, c):
                for j in range(D_MODEL // LANES):
                    sl = pl.ds(j * LANES, LANES)
                    buf_v[r, sl] = buf64_v[r, sl]
                return c

            lax.fori_loop(0, DCHUNK, widen_row, 0, unroll=4)
            pltpu.sync_copy(buf_v, wide_hbm.at[pl.ds(off, DCHUNK)])
            return carry

        def guarded(i, carry):
            @pl.when(wid + i * nw < v_blocks)
            def _():
                chunk_body(i, 0)

            return carry

        lax.fori_loop(0, v_iters, guarded, 0)

    @functools.partial(
        pl.kernel,
        mesh=mesh,
        out_type=jax.ShapeDtypeStruct((n, D_MODEL), jnp.float32),
        scratch_types=[
            pltpu.VMEM((CHUNK,), jnp.int32),
            pltpu.VMEM((CHUNK, 128), jnp.float32),
            pltpu.VMEM((CHUNK, D_MODEL), jnp.float32),
            pltpu.SemaphoreType.DMA,
        ],
    )
    def gather(x_hbm, wide_hbm, out_hbm, idx_v, rows_v, outbuf_v, sem):
        wid = lax.axis_index("s") * info.num_cores + lax.axis_index("c")
        base = wid * n_per_w

        def chunk_body(i, carry):
            off = pl.multiple_of(base + i * CHUNK, 8)
            pltpu.sync_copy(x_hbm.at[pl.ds(off, CHUNK)], idx_v)
            pltpu.async_copy(wide_hbm.at[idx_v], rows_v, sem).wait()

            def scale_row(r, c):
                for j in range(D_MODEL // LANES):
                    sl = pl.ds(j * LANES, LANES)
                    outbuf_v[r, sl] = rows_v[r, sl] * SCALE
                return c

            lax.fori_loop(0, CHUNK, scale_row, 0, unroll=4)
            pltpu.sync_copy(outbuf_v, out_hbm.at[pl.ds(off, CHUNK)])
            return carry

        lax.fori_loop(0, n_chunks, chunk_body, 0)

    wide = detile(lut)
    return gather(xf, wide)


def kernel(x, lut):
    b, h = x.shape
    out = _emb_call(x.reshape(b * h), lut)
    return out.reshape(b, h, D_MODEL)


# speedup vs baseline: 1.2342x; 1.2342x over previous
import functools

import jax
import jax.numpy as jnp
from jax import lax
from jax.experimental import pallas as pl
from jax.experimental.pallas import tpu as pltpu
from jax.experimental.pallas import tpu_sc as plsc

D_MODEL = 64
SCALE = 8.0
CHUNK = 512
TBLK = 512


def _xpose_body(lutT_ref, wide_ref):
    t = lutT_ref[...]
    tt = jnp.transpose(t, (1, 0)) * SCALE
    wide_ref[...] = jnp.concatenate([tt, tt], axis=1)


@jax.jit
def _emb_call(xf, lutT):
    v = lutT.shape[1]
    n = xf.shape[0]
    grid = (v + TBLK - 1) // TBLK

    wide = pl.pallas_call(
        _xpose_body,
        grid=(grid,),
        in_specs=[pl.BlockSpec((D_MODEL, TBLK), lambda i: (0, i))],
        out_specs=pl.BlockSpec((TBLK, 128), lambda i: (i, 0)),
        out_shape=jax.ShapeDtypeStruct((v, 128), jnp.float32),
    )(lutT)

    info = plsc.get_sparse_core_info()
    nw = info.num_cores * info.num_subcores
    n_per_w = n // nw
    n_chunks = n_per_w // CHUNK

    mesh = plsc.VectorSubcoreMesh(core_axis_name="c", subcore_axis_name="s")

    @functools.partial(
        pl.kernel,
        mesh=mesh,
        out_type=jax.ShapeDtypeStruct((n, 128), jnp.float32),
        scratch_types=[
            pltpu.VMEM((CHUNK,), jnp.int32),
            pltpu.VMEM((CHUNK, 128), jnp.float32),
            pltpu.SemaphoreType.DMA,
        ],
    )
    def gather(x_hbm, wide_hbm, out_hbm, idx_v, rows_v, sem):
        wid = lax.axis_index("s") * info.num_cores + lax.axis_index("c")
        base = wid * n_per_w

        def chunk_body(i, carry):
            off = pl.multiple_of(base + i * CHUNK, 8)
            pltpu.sync_copy(x_hbm.at[pl.ds(off, CHUNK)], idx_v)
            pltpu.async_copy(wide_hbm.at[idx_v], rows_v, sem).wait()
            pltpu.sync_copy(rows_v, out_hbm.at[pl.ds(off, CHUNK)])
            return carry

        lax.fori_loop(0, n_chunks, chunk_body, 0)

    return gather(xf, wide)


def kernel(x, lut):
    b, h = x.shape
    out128 = _emb_call(x.reshape(b * h), lut.T)
    return out128[:, :D_MODEL].reshape(b, h, D_MODEL)


# speedup vs baseline: 2.2207x; 1.7993x over previous
import functools

import jax
import jax.numpy as jnp
from jax import lax
from jax.experimental import pallas as pl
from jax.experimental.pallas import tpu as pltpu
from jax.experimental.pallas import tpu_sc as plsc

D_MODEL = 64
SCALE = 8.0
CHUNK = 512
TBLK = 2048


def _xpose_body(lutT_ref, wide_ref):
    t = lutT_ref[...]
    tt = jnp.transpose(t, (1, 0)) * SCALE
    wide_ref[...] = jnp.pad(tt, ((0, 0), (0, 128 - D_MODEL)))


@jax.jit
def _emb_call(xf, lutT):
    v = lutT.shape[1]
    n = xf.shape[0]
    grid = (v + TBLK - 1) // TBLK

    wide = pl.pallas_call(
        _xpose_body,
        grid=(grid,),
        in_specs=[pl.BlockSpec((D_MODEL, TBLK), lambda i: (0, i))],
        out_specs=pl.BlockSpec((TBLK, 128), lambda i: (i, 0)),
        out_shape=jax.ShapeDtypeStruct((v, 128), jnp.float32),
    )(lutT)

    info = plsc.get_sparse_core_info()
    nw = info.num_cores * info.num_subcores
    n_per_w = n // nw
    n_chunks = n_per_w // CHUNK

    mesh = plsc.VectorSubcoreMesh(core_axis_name="c", subcore_axis_name="s")

    @functools.partial(
        pl.kernel,
        mesh=mesh,
        out_type=jax.ShapeDtypeStruct((n, 128), jnp.float32),
        scratch_types=[
            pltpu.VMEM((CHUNK,), jnp.int32),
            pltpu.VMEM((CHUNK, 128), jnp.float32),
            pltpu.SemaphoreType.DMA,
        ],
    )
    def gather(x_hbm, wide_hbm, out_hbm, idx_v, rows_v, sem):
        wid = lax.axis_index("s") * info.num_cores + lax.axis_index("c")
        base = wid * n_per_w

        def chunk_body(i, carry):
            off = pl.multiple_of(base + i * CHUNK, 8)
            pltpu.sync_copy(x_hbm.at[pl.ds(off, CHUNK)], idx_v)
            pltpu.async_copy(wide_hbm.at[idx_v], rows_v, sem).wait()
            pltpu.sync_copy(rows_v, out_hbm.at[pl.ds(off, CHUNK)])
            return carry

        lax.fori_loop(0, n_chunks, chunk_body, 0)

    return gather(xf, wide)


def kernel(x, lut):
    b, h = x.shape
    out128 = _emb_call(x.reshape(b * h), lut.T)
    return out128[:, :D_MODEL].reshape(b, h, D_MODEL)


# speedup vs baseline: 2.7774x; 1.2507x over previous
import functools

import jax
import jax.numpy as jnp
from jax import lax
from jax.experimental import pallas as pl
from jax.experimental.pallas import tpu as pltpu
from jax.experimental.pallas import tpu_sc as plsc

D_MODEL = 64
SCALE = 8.0
CHUNK = 512
TBLK = 8192


def _xpose_body(lutT_ref, wide_ref):
    t = lutT_ref[...]
    tt = jnp.transpose(t, (1, 0)) * SCALE
    wide_ref[...] = jnp.pad(tt, ((0, 0), (0, 128 - D_MODEL)))


@jax.jit
def _emb_call(xf, lutT):
    v = lutT.shape[1]
    n = xf.shape[0]
    grid = (v + TBLK - 1) // TBLK

    wide = pl.pallas_call(
        _xpose_body,
        grid=(grid,),
        in_specs=[pl.BlockSpec((D_MODEL, TBLK), lambda i: (0, i))],
        out_specs=pl.BlockSpec((TBLK, 128), lambda i: (i, 0)),
        out_shape=jax.ShapeDtypeStruct((v, 128), jnp.float32),
    )(lutT)

    info = plsc.get_sparse_core_info()
    nw = info.num_cores * info.num_subcores
    n_per_w = n // nw
    n_chunks = n_per_w // CHUNK

    mesh = plsc.VectorSubcoreMesh(core_axis_name="c", subcore_axis_name="s")

    @functools.partial(
        pl.kernel,
        mesh=mesh,
        out_type=jax.ShapeDtypeStruct((n, 128), jnp.float32),
        scratch_types=[
            pltpu.VMEM((CHUNK,), jnp.int32),
            pltpu.VMEM((CHUNK, 128), jnp.float32),
            pltpu.SemaphoreType.DMA,
        ],
    )
    def gather(x_hbm, wide_hbm, out_hbm, idx_v, rows_v, sem):
        wid = lax.axis_index("s") * info.num_cores + lax.axis_index("c")
        base = wid * n_per_w

        def chunk_body(i, carry):
            off = pl.multiple_of(base + i * CHUNK, 8)
            pltpu.sync_copy(x_hbm.at[pl.ds(off, CHUNK)], idx_v)
            pltpu.async_copy(wide_hbm.at[idx_v], rows_v, sem).wait()
            pltpu.sync_copy(rows_v, out_hbm.at[pl.ds(off, CHUNK)])
            return carry

        lax.fori_loop(0, n_chunks, chunk_body, 0)

    return gather(xf, wide)


def kernel(x, lut):
    b, h = x.shape
    out128 = _emb_call(x.reshape(b * h), lut.T)
    return out128[:, :D_MODEL].reshape(b, h, D_MODEL)


# speedup vs baseline: 2.8413x; 1.0230x over previous
import functools

import jax
import jax.numpy as jnp
from jax import lax
from jax.experimental import pallas as pl
from jax.experimental.pallas import tpu as pltpu
from jax.experimental.pallas import tpu_sc as plsc

D_MODEL = 64
SCALE = 8.0
CHUNK = 512
TBLK = 16384


def _xpose_body(lutT_ref, wide_ref):
    t = lutT_ref[...]
    tt = jnp.transpose(t, (1, 0)) * SCALE
    wide_ref[...] = jnp.pad(tt, ((0, 0), (0, 128 - D_MODEL)))


@jax.jit
def _emb_call(xf, lutT):
    v = lutT.shape[1]
    n = xf.shape[0]
    grid = (v + TBLK - 1) // TBLK

    wide = pl.pallas_call(
        _xpose_body,
        grid=(grid,),
        in_specs=[pl.BlockSpec((D_MODEL, TBLK), lambda i: (0, i))],
        out_specs=pl.BlockSpec((TBLK, 128), lambda i: (i, 0)),
        out_shape=jax.ShapeDtypeStruct((v, 128), jnp.float32),
    )(lutT)

    info = plsc.get_sparse_core_info()
    nw = info.num_cores * info.num_subcores
    n_per_w = n // nw
    n_chunks = n_per_w // CHUNK

    mesh = plsc.VectorSubcoreMesh(core_axis_name="c", subcore_axis_name="s")

    @functools.partial(
        pl.kernel,
        mesh=mesh,
        out_type=jax.ShapeDtypeStruct((n, 128), jnp.float32),
        scratch_types=[
            pltpu.VMEM((CHUNK,), jnp.int32),
            pltpu.VMEM((CHUNK, 128), jnp.float32),
            pltpu.SemaphoreType.DMA,
        ],
    )
    def gather(x_hbm, wide_hbm, out_hbm, idx_v, rows_v, sem):
        wid = lax.axis_index("s") * info.num_cores + lax.axis_index("c")
        base = wid * n_per_w

        def chunk_body(i, carry):
            off = pl.multiple_of(base + i * CHUNK, 8)
            pltpu.sync_copy(x_hbm.at[pl.ds(off, CHUNK)], idx_v)
            pltpu.async_copy(wide_hbm.at[idx_v], rows_v, sem).wait()
            pltpu.sync_copy(rows_v, out_hbm.at[pl.ds(off, CHUNK)])
            return carry

        lax.fori_loop(0, n_chunks, chunk_body, 0)

    return gather(xf, wide)


def kernel(x, lut):
    b, h = x.shape
    out128 = _emb_call(x.reshape(b * h), lut.T)
    return out128[:, :D_MODEL].reshape(b, h, D_MODEL)


# speedup vs baseline: 2.8641x; 1.0080x over previous
import functools

import jax
import jax.numpy as jnp
from jax import lax
from jax.experimental import pallas as pl
from jax.experimental.pallas import tpu as pltpu
from jax.experimental.pallas import tpu_sc as plsc

D_MODEL = 64
SCALE = 8.0
CHUNK = 512
TBLK = 32768


def _xpose_body(lutT_ref, wide_ref):
    t = lutT_ref[...]
    tt = jnp.transpose(t, (1, 0)) * SCALE
    wide_ref[...] = jnp.pad(tt, ((0, 0), (0, 128 - D_MODEL)))


@jax.jit
def _emb_call(xf, lutT):
    v = lutT.shape[1]
    n = xf.shape[0]
    grid = (v + TBLK - 1) // TBLK

    wide = pl.pallas_call(
        _xpose_body,
        grid=(grid,),
        in_specs=[pl.BlockSpec((D_MODEL, TBLK), lambda i: (0, i))],
        out_specs=pl.BlockSpec((TBLK, 128), lambda i: (i, 0)),
        out_shape=jax.ShapeDtypeStruct((v, 128), jnp.float32),
    )(lutT)

    info = plsc.get_sparse_core_info()
    nw = info.num_cores * info.num_subcores
    n_per_w = n // nw
    n_chunks = n_per_w // CHUNK

    mesh = plsc.VectorSubcoreMesh(core_axis_name="c", subcore_axis_name="s")

    @functools.partial(
        pl.kernel,
        mesh=mesh,
        out_type=jax.ShapeDtypeStruct((n, 128), jnp.float32),
        scratch_types=[
            pltpu.VMEM((CHUNK,), jnp.int32),
            pltpu.VMEM((CHUNK, 128), jnp.float32),
            pltpu.SemaphoreType.DMA,
        ],
    )
    def gather(x_hbm, wide_hbm, out_hbm, idx_v, rows_v, sem):
        wid = lax.axis_index("s") * info.num_cores + lax.axis_index("c")
        base = wid * n_per_w

        def chunk_body(i, carry):
            off = pl.multiple_of(base + i * CHUNK, 8)
            pltpu.sync_copy(x_hbm.at[pl.ds(off, CHUNK)], idx_v)
            pltpu.async_copy(wide_hbm.at[idx_v], rows_v, sem).wait()
            pltpu.sync_copy(rows_v, out_hbm.at[pl.ds(off, CHUNK)])
            return carry

        lax.fori_loop(0, n_chunks, chunk_body, 0)

    return gather(xf, wide)


def kernel(x, lut):
    b, h = x.shape
    out128 = _emb_call(x.reshape(b * h), lut.T)
    return out128[:, :D_MODEL].reshape(b, h, D_MODEL)
